# Initial kernel scaffold; baseline (speedup 1.0000x reference)
import jax
import jax.numpy as jnp
from jax.experimental import pallas as pl


def kernel(x, pos, edge_index, Wm, bm, Wu, bu):
    raise NotImplementedError("write your pallas kernel here")


# speedup vs baseline: 2.9663x; 2.9663x over previous
import functools

import jax
import jax.numpy as jnp
from jax import lax
from jax.experimental import pallas as pl
from jax.experimental.pallas import tpu as pltpu
from jax.experimental.pallas import tpu_sc as plsc

N_NODES = 10000
N_EDGES = 320000
DIM = 128

NC = 2
NS = 16
NW = NC * NS

N_PAD = 10240
E_PAD = 327680
EPW = E_PAD // NW
ROWS_PW = EPW // 128
STRIPE = N_PAD // NS

_mesh = plsc.VectorSubcoreMesh(core_axis_name="c", subcore_axis_name="s")


def _worker_id():
    return lax.axis_index("s") * NC + lax.axis_index("c")


def _gather_body(a_hbm, b_hbm, row_hbm, col_hbm, e1_hbm, e2_hbm,
                 idxr, idxc, bufa, bufb, sema, semb, semo):
    w = _worker_id()

    def chunk(k, carry):
        rbase = w * ROWS_PW + k
        ebase = w * EPW + k * 128
        @pl.when(k > 0)
        def _drain():
            pltpu.make_async_copy(bufa, e1_hbm.at[pl.ds(0, 128)], semo).wait()
            pltpu.make_async_copy(bufb, e2_hbm.at[pl.ds(0, 128)], semo).wait()

        pltpu.sync_copy(row_hbm.at[pl.ds(rbase, 1)], idxr)
        pltpu.sync_copy(col_hbm.at[pl.ds(rbase, 1)], idxc)
        ca = pltpu.async_copy(a_hbm.at[idxr.at[0]], bufa, sema)
        cb = pltpu.async_copy(b_hbm.at[idxc.at[0]], bufb, semb)
        ca.wait()
        cb.wait()
        pltpu.async_copy(bufa, e1_hbm.at[pl.ds(ebase, 128)], semo)
        pltpu.async_copy(bufb, e2_hbm.at[pl.ds(ebase, 128)], semo)
        return carry

    lax.fori_loop(0, ROWS_PW, chunk, 0)
    pltpu.make_async_copy(bufa, e1_hbm.at[pl.ds(0, 128)], semo).wait()
    pltpu.make_async_copy(bufb, e2_hbm.at[pl.ds(0, 128)], semo).wait()


_gather_call = pl.kernel(
    _gather_body,
    out_type=[jax.ShapeDtypeStruct((E_PAD, DIM), jnp.float32),
              jax.ShapeDtypeStruct((E_PAD, DIM), jnp.float32)],
    mesh=_mesh,
    scratch_types=[
        pltpu.VMEM((1, 128), jnp.int32),
        pltpu.VMEM((1, 128), jnp.int32),
        pltpu.VMEM((128, DIM), jnp.float32),
        pltpu.VMEM((128, DIM), jnp.float32),
        pltpu.SemaphoreType.DMA,
        pltpu.SemaphoreType.DMA,
        pltpu.SemaphoreType.DMA,
    ],
)


N_HALF = 5120
H_PAD = 6144
HSTRIPE = H_PAD // NS
NZC = HSTRIPE // 128
ROWS_PT = E_PAD // 128 // NS
EPT = E_PAD // NS


def _fill_stripe_indices(zidx, base):
    lane = lax.iota(jnp.int32, 16)
    for j in range(NZC):
        for l in range(8):
            zidx[j, pl.ds(l * 16, 16)] = lane + (base + j * 128 + l * 16)


def _scatter_body(m_hbm, col_hbm, out_hbm, buf, idxc, idxl, zidx, acc, sem):
    cid = lax.axis_index("c")
    sid = lax.axis_index("s")
    nbase = cid * N_HALF

    def zrow(i, carry):
        for l in range(DIM // 16):
            buf[i, pl.ds(l * 16, 16)] = jnp.zeros((16,), jnp.float32)
        return carry

    lax.fori_loop(0, HSTRIPE, zrow, 0)
    _fill_stripe_indices(zidx, sid * HSTRIPE)
    for j in range(NZC):
        pltpu.sync_copy(buf.at[pl.ds(j * 128, 128)], acc.at[zidx.at[j]])
    plsc.subcore_barrier()

    def chunk(k, carry):
        rbase = sid * ROWS_PT + k
        ebase = sid * EPT + k * 128
        pltpu.sync_copy(col_hbm.at[pl.ds(rbase, 1)], idxc)
        pltpu.sync_copy(m_hbm.at[pl.ds(ebase, 128)], buf.at[pl.ds(0, 128)])
        for l in range(8):
            v = idxc[0, pl.ds(l * 16, 16)]
            loc = v - nbase
            ok = (loc >= 0) & (loc < N_HALF)
            idxl[0, pl.ds(l * 16, 16)] = jnp.where(
                ok, loc, N_HALF + (v & (H_PAD - N_HALF - 1)))
        pltpu.sync_copy(buf.at[pl.ds(0, 128)], acc.at[idxl.at[0]], add=True)
        return carry

    lax.fori_loop(0, ROWS_PT, chunk, 0)
    plsc.subcore_barrier()
    for j in range(NZC):
        pltpu.sync_copy(acc.at[zidx.at[j]], buf.at[pl.ds(j * 128, 128)])
    pltpu.sync_copy(buf, out_hbm.at[cid, pl.ds(sid * HSTRIPE, HSTRIPE)])


_scatter_call = pl.kernel(
    _scatter_body,
    out_type=jax.ShapeDtypeStruct((NC, H_PAD, DIM), jnp.float32),
    mesh=_mesh,
    scratch_types=[
        pltpu.VMEM((HSTRIPE, DIM), jnp.float32),
        pltpu.VMEM((1, 128), jnp.int32),
        pltpu.VMEM((1, 128), jnp.int32),
        pltpu.VMEM((NZC, 128), jnp.int32),
        pltpu.VMEM_SHARED((H_PAD, DIM), jnp.float32),
        pltpu.SemaphoreType.DMA,
    ],
)


def _layernorm(v, eps=1e-5):
    mu = jnp.mean(v, axis=-1, keepdims=True)
    var = jnp.mean((v - mu) ** 2, axis=-1, keepdims=True)
    return (v - mu) * lax.rsqrt(var + eps)


def _gelu(v):
    return 0.5 * v * (1.0 + lax.erf(v * (2.0 ** -0.5)))


def _ab_kernel(h_ref, pp_ref, w1_ref, w2_ref, w3_ref, bm_ref, a_ref, b_ref):
    h = h_ref[...]
    p3 = jnp.dot(pp_ref[...], w3_ref[...], preferred_element_type=jnp.float32)
    a_ref[...] = (jnp.dot(h, w1_ref[...], preferred_element_type=jnp.float32)
                  + p3 + bm_ref[...])
    b_ref[...] = (jnp.dot(h, w2_ref[...], preferred_element_type=jnp.float32)
                  - p3)


def _ab_call(h, posp, w1, w2, w3p, bm2):
    blk = 1000
    grid = N_NODES // blk
    return pl.pallas_call(
        _ab_kernel,
        grid=(grid,),
        in_specs=[
            pl.BlockSpec((blk, DIM), lambda i: (i, 0)),
            pl.BlockSpec((blk, 8), lambda i: (i, 0)),
            pl.BlockSpec((DIM, DIM), lambda i: (0, 0)),
            pl.BlockSpec((DIM, DIM), lambda i: (0, 0)),
            pl.BlockSpec((8, DIM), lambda i: (0, 0)),
            pl.BlockSpec((1, DIM), lambda i: (0, 0)),
        ],
        out_specs=[pl.BlockSpec((blk, DIM), lambda i: (i, 0)),
                   pl.BlockSpec((blk, DIM), lambda i: (i, 0))],
        out_shape=[jax.ShapeDtypeStruct((N_NODES, DIM), jnp.float32),
                   jax.ShapeDtypeStruct((N_NODES, DIM), jnp.float32)],
    )(h, posp, w1, w2, w3p, bm2)


def _msg_kernel(e1_ref, e2_ref, m_ref):
    m_ref[...] = _layernorm(_gelu(e1_ref[...] + e2_ref[...]))


def _msg_call(e1, e2):
    blk = 2048
    grid = E_PAD // blk
    return pl.pallas_call(
        _msg_kernel,
        grid=(grid,),
        in_specs=[pl.BlockSpec((blk, DIM), lambda i: (i, 0)),
                  pl.BlockSpec((blk, DIM), lambda i: (i, 0))],
        out_specs=pl.BlockSpec((blk, DIM), lambda i: (i, 0)),
        out_shape=jax.ShapeDtypeStruct((E_PAD, DIM), jnp.float32),
    )(e1, e2)


def _upd_kernel(h_ref, s_ref, c_ref, wu1_ref, wu2_ref, bu_ref, o_ref):
    h = h_ref[...]
    cnt = c_ref[:, 0:1]
    mean = s_ref[...] * (1.0 / jnp.maximum(cnt, 1.0))
    u = (jnp.dot(h, wu1_ref[...], preferred_element_type=jnp.float32)
         + jnp.dot(mean, wu2_ref[...], preferred_element_type=jnp.float32)
         + bu_ref[...])
    o_ref[...] = h + _layernorm(u)


def _upd_call(h, s, cnt, wu1, wu2, bu2):
    blk = 1000
    grid = N_NODES // blk
    return pl.pallas_call(
        _upd_kernel,
        grid=(grid,),
        in_specs=[
            pl.BlockSpec((blk, DIM), lambda i: (i, 0)),
            pl.BlockSpec((blk, DIM), lambda i: (i, 0)),
            pl.BlockSpec((blk, DIM), lambda i: (i, 0)),
            pl.BlockSpec((DIM, DIM), lambda i: (0, 0)),
            pl.BlockSpec((DIM, DIM), lambda i: (0, 0)),
            pl.BlockSpec((1, DIM), lambda i: (0, 0)),
        ],
        out_specs=pl.BlockSpec((blk, DIM), lambda i: (i, 0)),
        out_shape=jax.ShapeDtypeStruct((N_NODES, DIM), jnp.float32),
    )(h, s, cnt, wu1, wu2, bu2)


def kernel(x, pos, edge_index, Wm, bm, Wu, bu):
    row = edge_index[0].astype(jnp.int32)
    col = edge_index[1].astype(jnp.int32)
    npad = E_PAD - N_EDGES
    pad_ar = jnp.arange(npad, dtype=jnp.int32)
    rowp = jnp.concatenate([row, pad_ar % N_NODES]).reshape(E_PAD // 128, 128)
    colp = jnp.concatenate(
        [col, N_NODES + pad_ar % (N_PAD - N_NODES)]).reshape(E_PAD // 128, 128)

    posp = jnp.pad(pos.astype(jnp.float32), ((0, 0), (0, 8 - pos.shape[1])))

    def _seg_sum(m):
        sraw = _scatter_call(m, colp)
        return jnp.concatenate([sraw[0, :N_HALF], sraw[1, :N_HALF]], axis=0)

    cnt = _seg_sum(jnp.ones((E_PAD, DIM), jnp.float32))
    h, cnt = lax.optimization_barrier((x, cnt))
    for i in range(3):
        w1 = Wm[i, :DIM]
        w2 = Wm[i, DIM:2 * DIM]
        w3p = jnp.pad(Wm[i, 2 * DIM:], ((0, 8 - (Wm.shape[1] - 2 * DIM)), (0, 0)))
        a, b = _ab_call(h, posp, w1, w2, w3p, bm[i][None])
        e1, e2 = _gather_call(a, b, rowp, colp)
        m = _msg_call(e1, e2)
        s = _seg_sum(m)
        h = _upd_call(h, s, cnt, Wu[i, :DIM], Wu[i, DIM:], bu[i][None])
    return h
